# baseline (device time: 10977 ns/iter reference)
import jax
import jax.numpy as jnp
from jax import lax
from jax.experimental import pallas as pl
from jax.experimental.pallas import tpu as pltpu

N_DEV = 4
CAP = 158


def kernel(x, dest):
    n, d = x.shape
    drows = n // d

    def body(x_hbm, dest_hbm, out_hbm, pay_ref, recv_ref, dpay_ref, drecv_ref,
             x_vmem, d_vmem, o_vmem,
             send_sems, recv_sems, dsend_sems, drecv_sems, local_sem, io_sems):
        me = lax.axis_index("i")

        d_in = pltpu.make_async_copy(dest_hbm, d_vmem, io_sems.at[0])
        d_in.start()
        x_in = pltpu.make_async_copy(x_hbm, x_vmem, io_sems.at[1])
        x_in.start()

        barrier = pltpu.get_barrier_semaphore()
        for k in range(1, N_DEV):
            pl.semaphore_signal(
                barrier, inc=1,
                device_id=((me + k) % N_DEV,),
                device_id_type=pl.DeviceIdType.MESH,
            )
        pl.semaphore_wait(barrier, N_DEV - 1)

        d_in.wait()
        d_row = d_vmem[...]
        d_bf = d_row.astype(jnp.bfloat16)
        dpay_ref[0:1, :] = d_bf[:, :d]
        dpay_ref[1:2, :] = d_bf[:, d:]
        for r in range(N_DEV):
            @pl.when(r == me)
            def _():
                drecv_ref[r, 0:1, :] = d_bf[:, :d]
                drecv_ref[r, 1:2, :] = d_bf[:, d:]
        ddmas = []
        for k in range(1, N_DEV):
            p = (me + k) % N_DEV
            ddma = pltpu.make_async_remote_copy(
                src_ref=dpay_ref,
                dst_ref=drecv_ref.at[me],
                send_sem=dsend_sems.at[p],
                recv_sem=drecv_sems.at[me],
                device_id=(p,),
                device_id_type=pl.DeviceIdType.MESH,
            )
            ddma.start()
            ddmas.append(ddma)

        d_bcast = jnp.broadcast_to(d_row, (N_DEV, n))
        r_iota = lax.broadcasted_iota(jnp.int32, (N_DEV, n), 0)
        member = (d_bcast == r_iota).astype(jnp.float32)
        csum = member
        sh = 1
        while sh < n:
            csum = csum + jnp.concatenate(
                [jnp.zeros((N_DEV, sh), jnp.float32), csum[:, :-sh]], axis=1
            )
            sh *= 2
        ranks = csum - member
        key = jnp.where(member > 0.0, ranks, -1.0)

        x_in.wait()
        x_val = x_vmem[...]
        q_iota = lax.broadcasted_iota(jnp.int32, (CAP, n), 0).astype(jnp.float32)
        local_copy = pltpu.make_async_copy(
            pay_ref.at[me], recv_ref.at[me], local_sem
        )
        for r in range(N_DEV):
            p_r = (key[r : r + 1, :] == q_iota).astype(jnp.float32)
            pay_ref[r] = lax.dot_general(
                p_r, x_val, (((1,), (0,)), ((), ())),
                preferred_element_type=jnp.float32,
            ).astype(jnp.bfloat16)

            rdma = pltpu.make_async_remote_copy(
                src_ref=pay_ref.at[r],
                dst_ref=recv_ref.at[me],
                send_sem=send_sems.at[r],
                recv_sem=recv_sems.at[me],
                device_id=(r,),
                device_id_type=pl.DeviceIdType.MESH,
            )

            @pl.when(r != me)
            def _():
                rdma.start()

            @pl.when(r == me)
            def _():
                local_copy.start()

        me_bf = me.astype(jnp.bfloat16)
        diff = (
            lax.broadcasted_iota(jnp.int32, (n, CAP), 0)
            - lax.broadcasted_iota(jnp.int32, (n, CAP), 1)
        ).astype(jnp.float32)
        off = jnp.float32(0.0)
        q_masks = []
        for s in range(N_DEV):
            dwaiter = pltpu.make_async_remote_copy(
                src_ref=dpay_ref,
                dst_ref=drecv_ref.at[s],
                send_sem=dsend_sems.at[s],
                recv_sem=drecv_sems.at[s],
                device_id=(s,),
                device_id_type=pl.DeviceIdType.MESH,
            )

            @pl.when(s != me)
            def _():
                dwaiter.wait_recv()

            cin_s = jnp.sum(
                (drecv_ref[s] == me_bf).astype(jnp.float32)
            )
            q_masks.append(
                (diff == off).astype(jnp.bfloat16)
            )
            off += cin_s

        acc = jnp.zeros((n, d), jnp.float32)
        for s in range(N_DEV):
            waiter = pltpu.make_async_remote_copy(
                src_ref=pay_ref.at[s],
                dst_ref=recv_ref.at[s],
                send_sem=send_sems.at[s],
                recv_sem=recv_sems.at[s],
                device_id=(s,),
                device_id_type=pl.DeviceIdType.MESH,
            )

            @pl.when(s != me)
            def _():
                waiter.wait_recv()

            @pl.when(s == me)
            def _():
                local_copy.wait()

            acc += lax.dot_general(
                q_masks[s], recv_ref[s], (((1,), (0,)), ((), ())),
                preferred_element_type=jnp.float32,
            )
        o_vmem[...] = acc
        out_copy = pltpu.make_async_copy(o_vmem, out_hbm, io_sems.at[2])
        out_copy.start()

        for r in range(N_DEV):
            for src, dst, ssem, rsem in (
                (pay_ref.at[r], recv_ref.at[r], send_sems, recv_sems),
                (dpay_ref, drecv_ref.at[r], dsend_sems, drecv_sems),
            ):
                drain = pltpu.make_async_remote_copy(
                    src_ref=src, dst_ref=dst,
                    send_sem=ssem.at[r], recv_sem=rsem.at[r],
                    device_id=(r,),
                    device_id_type=pl.DeviceIdType.MESH,
                )

                @pl.when(r != me)
                def _():
                    drain.wait_send()
        out_copy.wait()

    return pl.pallas_call(
        body,
        out_shape=jax.ShapeDtypeStruct((n, d), jnp.float32),
        in_specs=[
            pl.BlockSpec(memory_space=pl.ANY),
            pl.BlockSpec(memory_space=pl.ANY),
        ],
        out_specs=pl.BlockSpec(memory_space=pl.ANY),
        scratch_shapes=[
            pltpu.VMEM((N_DEV, CAP, d), jnp.bfloat16),
            pltpu.VMEM((N_DEV, CAP, d), jnp.bfloat16),
            pltpu.VMEM((drows, d), jnp.bfloat16),
            pltpu.VMEM((N_DEV, drows, d), jnp.bfloat16),
            pltpu.VMEM((n, d), jnp.float32),
            pltpu.VMEM((1, n), jnp.int32),
            pltpu.VMEM((n, d), jnp.float32),
            pltpu.SemaphoreType.DMA((N_DEV,)),
            pltpu.SemaphoreType.DMA((N_DEV,)),
            pltpu.SemaphoreType.DMA((N_DEV,)),
            pltpu.SemaphoreType.DMA((N_DEV,)),
            pltpu.SemaphoreType.DMA,
            pltpu.SemaphoreType.DMA((3,)),
        ],
        compiler_params=pltpu.CompilerParams(collective_id=0),
    )(x, dest.reshape(1, n))


# device time: 10891 ns/iter; 1.0079x vs baseline; 1.0079x over previous
import jax
import jax.numpy as jnp
from jax import lax
from jax.experimental import pallas as pl
from jax.experimental.pallas import tpu as pltpu

N_DEV = 4
CAP = 158


def kernel(x, dest):
    n, d = x.shape
    drows = n // d

    def body(x_ref, dest_ref, out_ref, pay_ref, recv_ref, dpay_ref, drecv_ref,
             send_sems, recv_sems, dsend_sems, drecv_sems, local_sem):
        me = lax.axis_index("i")
        x_val = x_ref[...]
        d_row = dest_ref[...]

        barrier = pltpu.get_barrier_semaphore()
        for k in range(1, N_DEV):
            pl.semaphore_signal(
                barrier, inc=1,
                device_id=((me + k) % N_DEV,),
                device_id_type=pl.DeviceIdType.MESH,
            )
        pl.semaphore_wait(barrier, N_DEV - 1)

        d_bf = d_row.astype(jnp.bfloat16)
        dpay_ref[0:1, :] = d_bf[:, :d]
        dpay_ref[1:2, :] = d_bf[:, d:]
        for r in range(N_DEV):
            @pl.when(r == me)
            def _():
                drecv_ref[r, 0:1, :] = d_bf[:, :d]
                drecv_ref[r, 1:2, :] = d_bf[:, d:]
        ddmas = []
        for k in range(1, N_DEV):
            p = (me + k) % N_DEV
            ddma = pltpu.make_async_remote_copy(
                src_ref=dpay_ref,
                dst_ref=drecv_ref.at[me],
                send_sem=dsend_sems.at[p],
                recv_sem=drecv_sems.at[me],
                device_id=(p,),
                device_id_type=pl.DeviceIdType.MESH,
            )
            ddma.start()
            ddmas.append(ddma)

        d_bcast = jnp.broadcast_to(d_row, (N_DEV, n))
        r_iota = lax.broadcasted_iota(jnp.int32, (N_DEV, n), 0)
        member = (d_bcast == r_iota).astype(jnp.float32)
        csum = member
        sh = 1
        while sh < n:
            csum = csum + jnp.concatenate(
                [jnp.zeros((N_DEV, sh), jnp.float32), csum[:, :-sh]], axis=1
            )
            sh *= 2
        ranks = csum - member
        key = jnp.where(member > 0.0, ranks, -1.0)

        q_iota = lax.broadcasted_iota(jnp.int32, (CAP, n), 0).astype(jnp.float32)
        local_copy = pltpu.make_async_copy(
            pay_ref.at[me], recv_ref.at[me], local_sem
        )
        for r in range(N_DEV):
            p_r = (key[r : r + 1, :] == q_iota).astype(jnp.float32)
            pay_ref[r] = lax.dot_general(
                p_r, x_val, (((1,), (0,)), ((), ())),
                preferred_element_type=jnp.float32,
            ).astype(jnp.bfloat16)

            rdma = pltpu.make_async_remote_copy(
                src_ref=pay_ref.at[r],
                dst_ref=recv_ref.at[me],
                send_sem=send_sems.at[r],
                recv_sem=recv_sems.at[me],
                device_id=(r,),
                device_id_type=pl.DeviceIdType.MESH,
            )

            @pl.when(r != me)
            def _():
                rdma.start()

            @pl.when(r == me)
            def _():
                local_copy.start()

        me_bf = me.astype(jnp.bfloat16)
        diff = (
            lax.broadcasted_iota(jnp.int32, (n, CAP), 0)
            - lax.broadcasted_iota(jnp.int32, (n, CAP), 1)
        ).astype(jnp.float32)
        for s in range(N_DEV):
            dwaiter = pltpu.make_async_remote_copy(
                src_ref=dpay_ref,
                dst_ref=drecv_ref.at[s],
                send_sem=dsend_sems.at[s],
                recv_sem=drecv_sems.at[s],
                device_id=(s,),
                device_id_type=pl.DeviceIdType.MESH,
            )

            @pl.when(s != me)
            def _():
                dwaiter.wait_recv()

        cnt = jnp.sum(
            (drecv_ref[...] == me_bf).astype(jnp.float32), axis=(1, 2)
        )
        off = jnp.float32(0.0)
        q_masks = []
        for s in range(N_DEV):
            q_masks.append(
                (diff == off).astype(jnp.bfloat16)
            )
            off += cnt[s]

        acc = jnp.zeros((n, d), jnp.float32)
        for s in range(N_DEV):
            waiter = pltpu.make_async_remote_copy(
                src_ref=pay_ref.at[s],
                dst_ref=recv_ref.at[s],
                send_sem=send_sems.at[s],
                recv_sem=recv_sems.at[s],
                device_id=(s,),
                device_id_type=pl.DeviceIdType.MESH,
            )

            @pl.when(s != me)
            def _():
                waiter.wait_recv()

            @pl.when(s == me)
            def _():
                local_copy.wait()

            acc += lax.dot_general(
                q_masks[s], recv_ref[s], (((1,), (0,)), ((), ())),
                preferred_element_type=jnp.float32,
            )
        out_ref[...] = acc

        for r in range(N_DEV):
            for src, dst, ssem, rsem in (
                (pay_ref.at[r], recv_ref.at[r], send_sems, recv_sems),
                (dpay_ref, drecv_ref.at[r], dsend_sems, drecv_sems),
            ):
                drain = pltpu.make_async_remote_copy(
                    src_ref=src, dst_ref=dst,
                    send_sem=ssem.at[r], recv_sem=rsem.at[r],
                    device_id=(r,),
                    device_id_type=pl.DeviceIdType.MESH,
                )

                @pl.when(r != me)
                def _():
                    drain.wait_send()

    return pl.pallas_call(
        body,
        out_shape=jax.ShapeDtypeStruct((n, d), jnp.float32),
        in_specs=[
            pl.BlockSpec(memory_space=pltpu.VMEM),
            pl.BlockSpec(memory_space=pltpu.VMEM),
        ],
        out_specs=pl.BlockSpec(memory_space=pltpu.VMEM),
        scratch_shapes=[
            pltpu.VMEM((N_DEV, CAP, d), jnp.bfloat16),
            pltpu.VMEM((N_DEV, CAP, d), jnp.bfloat16),
            pltpu.VMEM((drows, d), jnp.bfloat16),
            pltpu.VMEM((N_DEV, drows, d), jnp.bfloat16),
            pltpu.SemaphoreType.DMA((N_DEV,)),
            pltpu.SemaphoreType.DMA((N_DEV,)),
            pltpu.SemaphoreType.DMA((N_DEV,)),
            pltpu.SemaphoreType.DMA((N_DEV,)),
            pltpu.SemaphoreType.DMA,
        ],
        compiler_params=pltpu.CompilerParams(collective_id=0),
    )(x, dest.reshape(1, n))


# device time: 10258 ns/iter; 1.0701x vs baseline; 1.0617x over previous
import jax
import jax.numpy as jnp
from jax import lax
from jax.experimental import pallas as pl
from jax.experimental.pallas import tpu as pltpu

N_DEV = 4
CAP = 158


def kernel(x, dest):
    n, d = x.shape
    drows = n // d

    def body(x_ref, dest_ref, out_ref, pay_ref, recv_ref, dpay_ref, drecv_ref,
             send_sems, recv_sems, dsend_sems, drecv_sems, local_sem):
        me = lax.axis_index("i")
        x_val = x_ref[...]
        d_row = dest_ref[...]

        barrier = pltpu.get_barrier_semaphore()
        for k in range(1, N_DEV):
            pl.semaphore_signal(
                barrier, inc=1,
                device_id=((me + k) % N_DEV,),
                device_id_type=pl.DeviceIdType.MESH,
            )

        d_bf = d_row.astype(jnp.bfloat16)
        dpay_ref[0:1, :] = d_bf[:, :d]
        dpay_ref[1:2, :] = d_bf[:, d:]
        for r in range(N_DEV):
            @pl.when(r == me)
            def _():
                drecv_ref[r, 0:1, :] = d_bf[:, :d]
                drecv_ref[r, 1:2, :] = d_bf[:, d:]

        d_bcast = jnp.broadcast_to(d_row, (N_DEV, n))
        r_iota = lax.broadcasted_iota(jnp.int32, (N_DEV, n), 0)
        member = (d_bcast == r_iota).astype(jnp.float32)
        csum = member
        sh = 1
        while sh < n:
            csum = csum + jnp.concatenate(
                [jnp.zeros((N_DEV, sh), jnp.float32), csum[:, :-sh]], axis=1
            )
            sh *= 2
        ranks = csum - member
        key = jnp.where(member > 0.0, ranks, -1.0)

        q_iota = lax.broadcasted_iota(jnp.int32, (CAP, n), 0).astype(jnp.float32)
        local_copy = pltpu.make_async_copy(
            pay_ref.at[me], recv_ref.at[me], local_sem
        )
        for r in range(N_DEV):
            p_r = (key[r : r + 1, :] == q_iota).astype(jnp.float32)
            pay_ref[r] = lax.dot_general(
                p_r, x_val, (((1,), (0,)), ((), ())),
                preferred_element_type=jnp.float32,
            ).astype(jnp.bfloat16)

            @pl.when(r == me)
            def _():
                local_copy.start()

        pl.semaphore_wait(barrier, N_DEV - 1)
        for k in range(1, N_DEV):
            p = (me + k) % N_DEV
            pltpu.make_async_remote_copy(
                src_ref=dpay_ref,
                dst_ref=drecv_ref.at[me],
                send_sem=dsend_sems.at[p],
                recv_sem=drecv_sems.at[me],
                device_id=(p,),
                device_id_type=pl.DeviceIdType.MESH,
            ).start()
            pltpu.make_async_remote_copy(
                src_ref=pay_ref.at[p],
                dst_ref=recv_ref.at[me],
                send_sem=send_sems.at[p],
                recv_sem=recv_sems.at[me],
                device_id=(p,),
                device_id_type=pl.DeviceIdType.MESH,
            ).start()

        me_bf = me.astype(jnp.bfloat16)
        diff = (
            lax.broadcasted_iota(jnp.int32, (n, CAP), 0)
            - lax.broadcasted_iota(jnp.int32, (n, CAP), 1)
        ).astype(jnp.float32)
        for s in range(N_DEV):
            dwaiter = pltpu.make_async_remote_copy(
                src_ref=dpay_ref,
                dst_ref=drecv_ref.at[s],
                send_sem=dsend_sems.at[s],
                recv_sem=drecv_sems.at[s],
                device_id=(s,),
                device_id_type=pl.DeviceIdType.MESH,
            )

            @pl.when(s != me)
            def _():
                dwaiter.wait_recv()

        cnt = jnp.sum(
            (drecv_ref[...] == me_bf).astype(jnp.float32), axis=(1, 2)
        )
        off = jnp.float32(0.0)
        q_masks = []
        for s in range(N_DEV):
            q_masks.append(
                (diff == off).astype(jnp.bfloat16)
            )
            off += cnt[s]

        acc = jnp.zeros((n, d), jnp.float32)
        for s in range(N_DEV):
            waiter = pltpu.make_async_remote_copy(
                src_ref=pay_ref.at[s],
                dst_ref=recv_ref.at[s],
                send_sem=send_sems.at[s],
                recv_sem=recv_sems.at[s],
                device_id=(s,),
                device_id_type=pl.DeviceIdType.MESH,
            )

            @pl.when(s != me)
            def _():
                waiter.wait_recv()

            @pl.when(s == me)
            def _():
                local_copy.wait()

            acc += lax.dot_general(
                q_masks[s], recv_ref[s], (((1,), (0,)), ((), ())),
                preferred_element_type=jnp.float32,
            )
        out_ref[...] = acc

        for r in range(N_DEV):
            for src, dst, ssem, rsem in (
                (pay_ref.at[r], recv_ref.at[r], send_sems, recv_sems),
                (dpay_ref, drecv_ref.at[r], dsend_sems, drecv_sems),
            ):
                drain = pltpu.make_async_remote_copy(
                    src_ref=src, dst_ref=dst,
                    send_sem=ssem.at[r], recv_sem=rsem.at[r],
                    device_id=(r,),
                    device_id_type=pl.DeviceIdType.MESH,
                )

                @pl.when(r != me)
                def _():
                    drain.wait_send()

    return pl.pallas_call(
        body,
        out_shape=jax.ShapeDtypeStruct((n, d), jnp.float32),
        in_specs=[
            pl.BlockSpec(memory_space=pltpu.VMEM),
            pl.BlockSpec(memory_space=pltpu.VMEM),
        ],
        out_specs=pl.BlockSpec(memory_space=pltpu.VMEM),
        scratch_shapes=[
            pltpu.VMEM((N_DEV, CAP, d), jnp.bfloat16),
            pltpu.VMEM((N_DEV, CAP, d), jnp.bfloat16),
            pltpu.VMEM((drows, d), jnp.bfloat16),
            pltpu.VMEM((N_DEV, drows, d), jnp.bfloat16),
            pltpu.SemaphoreType.DMA((N_DEV,)),
            pltpu.SemaphoreType.DMA((N_DEV,)),
            pltpu.SemaphoreType.DMA((N_DEV,)),
            pltpu.SemaphoreType.DMA((N_DEV,)),
            pltpu.SemaphoreType.DMA,
        ],
        compiler_params=pltpu.CompilerParams(collective_id=0),
    )(x, dest.reshape(1, n))


# device time: 10107 ns/iter; 1.0861x vs baseline; 1.0149x over previous
import jax
import jax.numpy as jnp
from jax import lax
from jax.experimental import pallas as pl
from jax.experimental.pallas import tpu as pltpu

N_DEV = 4
CAP = 152


def kernel(x, dest):
    n, d = x.shape
    drows = n // d

    def body(x_ref, dest_ref, out_ref, pay_ref, recv_ref, dpay_ref, drecv_ref,
             send_sems, recv_sems, dsend_sems, drecv_sems, local_sem):
        me = lax.axis_index("i")
        x_val = x_ref[...]
        d_row = dest_ref[...]

        barrier = pltpu.get_barrier_semaphore()
        for k in range(1, N_DEV):
            pl.semaphore_signal(
                barrier, inc=1,
                device_id=((me + k) % N_DEV,),
                device_id_type=pl.DeviceIdType.MESH,
            )

        d_bf = d_row.astype(jnp.bfloat16)
        dpay_ref[0:1, :] = d_bf[:, :d]
        dpay_ref[1:2, :] = d_bf[:, d:]
        for r in range(N_DEV):
            @pl.when(r == me)
            def _():
                drecv_ref[r, 0:1, :] = d_bf[:, :d]
                drecv_ref[r, 1:2, :] = d_bf[:, d:]

        d_bcast = jnp.broadcast_to(d_row, (N_DEV, n))
        r_iota = lax.broadcasted_iota(jnp.int32, (N_DEV, n), 0)
        member = (d_bcast == r_iota).astype(jnp.float32)
        csum = member
        sh = 1
        while sh < n:
            csum = csum + jnp.concatenate(
                [jnp.zeros((N_DEV, sh), jnp.float32), csum[:, :-sh]], axis=1
            )
            sh *= 2
        ranks = csum - member
        key = jnp.where(member > 0.0, ranks, -1.0)

        q_iota = lax.broadcasted_iota(jnp.int32, (CAP, n), 0).astype(jnp.float32)
        local_copy = pltpu.make_async_copy(
            pay_ref.at[me], recv_ref.at[me], local_sem
        )
        for r in range(N_DEV):
            p_r = (key[r : r + 1, :] == q_iota).astype(jnp.float32)
            pay_ref[r] = lax.dot_general(
                p_r, x_val, (((1,), (0,)), ((), ())),
                preferred_element_type=jnp.float32,
            ).astype(jnp.bfloat16)

            @pl.when(r == me)
            def _():
                local_copy.start()

        pl.semaphore_wait(barrier, N_DEV - 1)
        for k in (2, 1, 3):
            p = (me + k) % N_DEV
            pltpu.make_async_remote_copy(
                src_ref=dpay_ref,
                dst_ref=drecv_ref.at[me],
                send_sem=dsend_sems.at[p],
                recv_sem=drecv_sems.at[me],
                device_id=(p,),
                device_id_type=pl.DeviceIdType.MESH,
            ).start()
            pltpu.make_async_remote_copy(
                src_ref=pay_ref.at[p],
                dst_ref=recv_ref.at[me],
                send_sem=send_sems.at[p],
                recv_sem=recv_sems.at[me],
                device_id=(p,),
                device_id_type=pl.DeviceIdType.MESH,
            ).start()

        me_bf = me.astype(jnp.bfloat16)
        diff = (
            lax.broadcasted_iota(jnp.int32, (n, CAP), 0)
            - lax.broadcasted_iota(jnp.int32, (n, CAP), 1)
        ).astype(jnp.float32)
        for s in range(N_DEV):
            dwaiter = pltpu.make_async_remote_copy(
                src_ref=dpay_ref,
                dst_ref=drecv_ref.at[s],
                send_sem=dsend_sems.at[s],
                recv_sem=drecv_sems.at[s],
                device_id=(s,),
                device_id_type=pl.DeviceIdType.MESH,
            )

            @pl.when(s != me)
            def _():
                dwaiter.wait_recv()

        cnt = jnp.sum(
            (drecv_ref[...] == me_bf).astype(jnp.float32), axis=(1, 2)
        )
        off = jnp.float32(0.0)
        q_masks = []
        for s in range(N_DEV):
            q_masks.append(
                (diff == off).astype(jnp.bfloat16)
            )
            off += cnt[s]

        acc = jnp.zeros((n, d), jnp.float32)
        for s in range(N_DEV):
            waiter = pltpu.make_async_remote_copy(
                src_ref=pay_ref.at[s],
                dst_ref=recv_ref.at[s],
                send_sem=send_sems.at[s],
                recv_sem=recv_sems.at[s],
                device_id=(s,),
                device_id_type=pl.DeviceIdType.MESH,
            )

            @pl.when(s != me)
            def _():
                waiter.wait_recv()

            @pl.when(s == me)
            def _():
                local_copy.wait()

            acc += lax.dot_general(
                q_masks[s], recv_ref[s], (((1,), (0,)), ((), ())),
                preferred_element_type=jnp.float32,
            )
        out_ref[...] = acc

        for r in range(N_DEV):
            for src, dst, ssem, rsem in (
                (pay_ref.at[r], recv_ref.at[r], send_sems, recv_sems),
                (dpay_ref, drecv_ref.at[r], dsend_sems, drecv_sems),
            ):
                drain = pltpu.make_async_remote_copy(
                    src_ref=src, dst_ref=dst,
                    send_sem=ssem.at[r], recv_sem=rsem.at[r],
                    device_id=(r,),
                    device_id_type=pl.DeviceIdType.MESH,
                )

                @pl.when(r != me)
                def _():
                    drain.wait_send()

    return pl.pallas_call(
        body,
        out_shape=jax.ShapeDtypeStruct((n, d), jnp.float32),
        in_specs=[
            pl.BlockSpec(memory_space=pltpu.VMEM),
            pl.BlockSpec(memory_space=pltpu.VMEM),
        ],
        out_specs=pl.BlockSpec(memory_space=pltpu.VMEM),
        scratch_shapes=[
            pltpu.VMEM((N_DEV, CAP, d), jnp.bfloat16),
            pltpu.VMEM((N_DEV, CAP, d), jnp.bfloat16),
            pltpu.VMEM((drows, d), jnp.bfloat16),
            pltpu.VMEM((N_DEV, drows, d), jnp.bfloat16),
            pltpu.SemaphoreType.DMA((N_DEV,)),
            pltpu.SemaphoreType.DMA((N_DEV,)),
            pltpu.SemaphoreType.DMA((N_DEV,)),
            pltpu.SemaphoreType.DMA((N_DEV,)),
            pltpu.SemaphoreType.DMA,
        ],
        compiler_params=pltpu.CompilerParams(collective_id=0),
    )(x, dest.reshape(1, n))
